# Initial kernel scaffold; baseline (speedup 1.0000x reference)
import jax
import jax.numpy as jnp
from jax.experimental import pallas as pl


def kernel(edge_rel, edge_src, edge_dst, edge_val, w_bases1, w_rel1, w_bases2, w_rel2):
    raise NotImplementedError("write your pallas kernel here")


# speedup vs baseline: 1.4065x; 1.4065x over previous
import dataclasses
import functools

import jax
import jax.numpy as jnp
from jax import lax
from jax.experimental import pallas as pl
from jax.experimental.pallas import tpu as pltpu
from jax.experimental.pallas import tpu_sc as plsc

N = 50000
R = 8
B = 2
H = 64
O = 64
E = 800000

NS = 16
BLK = 128
NBLK = E // BLK
KMAX = (NBLK + NS - 1) // NS
WBLK = 400
NWB = N // WBLK
KWB = (NWB + NS - 1) // NS


def _sc_pass(table, g, dst, val):
    mesh = plsc.VectorSubcoreMesh(core_axis_name="c", subcore_axis_name="s")

    cp = pltpu.CompilerParams()
    for f, v in (("needs_layout_passes", False), ("use_tc_tiling_on_sc", False)):
        if f in pltpu.CompilerParams.__dataclass_fields__:
            cp = dataclasses.replace(cp, **{f: v})

    @functools.partial(
        pl.kernel,
        compiler_params=cp,
        out_type=jax.ShapeDtypeStruct((2, N, 32), jnp.float32),
        mesh=mesh,
        scratch_types=[
            pltpu.VMEM((BLK,), jnp.int32),
            pltpu.VMEM((BLK,), jnp.int32),
            pltpu.VMEM((BLK,), jnp.float32),
            pltpu.VMEM((BLK, 32), jnp.float32),
            pltpu.VMEM((WBLK, 32), jnp.float32),
            pltpu.VMEM_SHARED((N, 32), jnp.float32),
        ],
    )
    def kern(table_hbm, g_hbm, dst_hbm, val_hbm, out_hbm,
             gv, dv, vv, rows, zb, acc):
        c = lax.axis_index("c")
        s = lax.axis_index("s")

        zeros16 = jnp.zeros((16,), jnp.float32)

        @pl.loop(0, WBLK)
        def _(i):
            zb[i, pl.ds(0, 16)] = zeros16
            zb[i, pl.ds(16, 16)] = zeros16

        @pl.loop(0, KWB)
        def _(k):
            b = k * NS + s

            @pl.when(b < NWB)
            def _():
                pltpu.sync_copy(zb, acc.at[pl.ds(b * WBLK, WBLK)])

        plsc.subcore_barrier()

        @pl.loop(0, KMAX)
        def _(k):
            b = k * NS + s

            @pl.when(b < NBLK)
            def _():
                base = b * BLK
                pltpu.sync_copy(g_hbm.at[pl.ds(base, BLK)], gv)
                pltpu.sync_copy(dst_hbm.at[pl.ds(base, BLK)], dv)
                pltpu.sync_copy(val_hbm.at[pl.ds(base, BLK)], vv)
                pltpu.sync_copy(table_hbm.at[c].at[gv], rows)

                @pl.loop(0, BLK)
                def _(e):
                    sv = plsc.load_gather(vv, [jnp.full((16,), e, jnp.int32)])
                    rows[e, pl.ds(0, 16)] = rows[e, pl.ds(0, 16)] * sv
                    rows[e, pl.ds(16, 16)] = rows[e, pl.ds(16, 16)] * sv

                pltpu.sync_copy(rows, acc.at[dv], add=True)

        plsc.subcore_barrier()

        @pl.loop(0, KWB)
        def _(k):
            b = k * NS + s

            @pl.when(b < NWB)
            def _():
                pltpu.sync_copy(
                    acc.at[pl.ds(b * WBLK, WBLK)],
                    out_hbm.at[c].at[pl.ds(b * WBLK, WBLK)],
                )

    return kern(table, g, dst, val)


TBLK = 400


def _build_t1(w_rel1, w_bases1):
    def body(wr_ref, wb_ref, out_ref):
        r = pl.program_id(0)
        blk = wr_ref[r, 0] * wb_ref[0] + wr_ref[r, 1] * wb_ref[1]
        out_ref[0, 0] = blk[:, :32]
        out_ref[1, 0] = blk[:, 32:]

    out = pl.pallas_call(
        body,
        grid=(R, N // TBLK),
        in_specs=[
            pl.BlockSpec(memory_space=pltpu.SMEM),
            pl.BlockSpec((B, TBLK, H), lambda r, i: (0, i, 0)),
        ],
        out_specs=pl.BlockSpec((2, 1, TBLK, 32), lambda r, i: (0, r, i, 0)),
        out_shape=jax.ShapeDtypeStruct((2, R, N, 32), jnp.float32),
    )(w_rel1, w_bases1)
    return out.reshape(2, R * N, 32)


def _build_t2(xh, w_rel2, w_bases2):
    def body(wr_ref, wb_ref, x_ref, out_ref):
        r = pl.program_id(1)
        xb = jnp.concatenate([x_ref[0], x_ref[1]], axis=1)
        xb = jnp.maximum(xb, 0.0)
        w2r = wr_ref[r, 0] * wb_ref[0] + wr_ref[r, 1] * wb_ref[1]
        y = jnp.dot(xb, w2r, preferred_element_type=jnp.float32,
                    precision=lax.Precision.HIGHEST)
        out_ref[0, 0] = y[:, :32]
        out_ref[1, 0] = y[:, 32:]

    out = pl.pallas_call(
        body,
        grid=(N // TBLK, R),
        in_specs=[
            pl.BlockSpec(memory_space=pltpu.SMEM),
            pl.BlockSpec((B, H, O), lambda i, r: (0, 0, 0)),
            pl.BlockSpec((2, TBLK, 32), lambda i, r: (0, i, 0)),
        ],
        out_specs=pl.BlockSpec((2, 1, TBLK, 32), lambda i, r: (0, r, i, 0)),
        out_shape=jax.ShapeDtypeStruct((2, R, N, 32), jnp.float32),
    )(w_rel2, w_bases2, xh)
    return out.reshape(2, R * N, 32)


@jax.jit
def kernel(edge_rel, edge_src, edge_dst, edge_val,
           w_bases1, w_rel1, w_bases2, w_rel2):
    edge_rel = edge_rel.astype(jnp.int32)
    edge_src = edge_src.astype(jnp.int32)
    edge_dst = edge_dst.astype(jnp.int32)

    g = edge_rel * N + edge_src

    t1 = _build_t1(w_rel1, w_bases1)
    xh = _sc_pass(t1, g, edge_dst, edge_val)
    t2 = _build_t2(xh, w_rel2, w_bases2)
    oh = _sc_pass(t2, g, edge_dst, edge_val)
    return jnp.concatenate([oh[0], oh[1]], axis=1)


# speedup vs baseline: 2.0724x; 1.4735x over previous
import dataclasses
import functools

import jax
import jax.numpy as jnp
from jax import lax
from jax.experimental import pallas as pl
from jax.experimental.pallas import tpu as pltpu
from jax.experimental.pallas import tpu_sc as plsc

N = 50000
R = 8
B = 2
H = 64
O = 64
E = 800000

NS = 16
IB = 100
IBN = 8
BLK = IB * IBN
NBLK = E // BLK
KMAX = (NBLK + NS - 1) // NS
MU = 8
WBLK = 400
NWB = N // WBLK
KWB = (NWB + NS - 1) // NS


def _sc_pass(table, g, dst, val):
    mesh = plsc.VectorSubcoreMesh(core_axis_name="c", subcore_axis_name="s")

    cp = pltpu.CompilerParams()
    for f, v in (("needs_layout_passes", False), ("use_tc_tiling_on_sc", False)):
        if f in pltpu.CompilerParams.__dataclass_fields__:
            cp = dataclasses.replace(cp, **{f: v})

    @functools.partial(
        pl.kernel,
        compiler_params=cp,
        out_type=jax.ShapeDtypeStruct((2, N, 32), jnp.float32),
        mesh=mesh,
        scratch_types=[
            pltpu.VMEM((IBN, IB), jnp.int32),
            pltpu.VMEM((IBN, IB), jnp.int32),
            pltpu.VMEM((BLK,), jnp.float32),
            pltpu.VMEM((BLK, 32), jnp.float32),
            pltpu.VMEM_SHARED((N, 32), jnp.float32),
            pltpu.SemaphoreType.DMA,
        ],
    )
    def kern(table_hbm, g_hbm, dst_hbm, val_hbm, out_hbm,
             gv, dv, vv, rows, acc, sem):
        c = lax.axis_index("c")
        s = lax.axis_index("s")

        zeros16 = jnp.zeros((16,), jnp.float32)

        @pl.loop(0, WBLK)
        def _(i):
            rows[i, pl.ds(0, 16)] = zeros16
            rows[i, pl.ds(16, 16)] = zeros16

        @pl.loop(0, KWB)
        def _(k):
            b = k * NS + s

            @pl.when(b < NWB)
            def _():
                pltpu.sync_copy(rows.at[pl.ds(0, WBLK)],
                                acc.at[pl.ds(b * WBLK, WBLK)])

        plsc.subcore_barrier()

        def _do_block(b):
            base = b * BLK
            c1 = pltpu.async_copy(g_hbm.at[b], gv, sem)
            c2 = pltpu.async_copy(dst_hbm.at[b], dv, sem)
            c3 = pltpu.async_copy(val_hbm.at[pl.ds(base, BLK)], vv, sem)
            c1.wait()
            c2.wait()
            c3.wait()

            gathers = [
                pltpu.async_copy(
                    table_hbm.at[c].at[gv.at[j]],
                    rows.at[pl.ds(j * IB, IB)], sem)
                for j in range(IBN)
            ]
            for g_ in gathers:
                g_.wait()

            @pl.loop(0, BLK, step=MU)
            def _(e0):
                for t in range(MU):
                    e = e0 + t
                    sv = plsc.load_gather(vv, [jnp.full((16,), e, jnp.int32)])
                    rows[e, pl.ds(0, 16)] = rows[e, pl.ds(0, 16)] * sv
                    rows[e, pl.ds(16, 16)] = rows[e, pl.ds(16, 16)] * sv

            scatters = [
                pltpu.async_copy(
                    rows.at[pl.ds(j * IB, IB)],
                    acc.at[dv.at[j]], sem, add=True)
                for j in range(IBN)
            ]
            for s_ in scatters:
                s_.wait()

        @pl.loop(0, KMAX)
        def _(k):
            b = k * NS + s

            @pl.when(b < NBLK)
            def _():
                _do_block(b)

        plsc.subcore_barrier()

        @pl.loop(0, KWB)
        def _(k):
            b = k * NS + s

            @pl.when(b < NWB)
            def _():
                pltpu.sync_copy(
                    acc.at[pl.ds(b * WBLK, WBLK)],
                    out_hbm.at[c].at[pl.ds(b * WBLK, WBLK)],
                )

    return kern(table, g, dst, val)


TBLK = 400


def _build_t1(w_rel1, w_bases1):
    def body(wr_ref, wb_ref, out_ref):
        r = pl.program_id(0)
        blk = wr_ref[r, 0] * wb_ref[0] + wr_ref[r, 1] * wb_ref[1]
        out_ref[0, 0] = blk[:, :32]
        out_ref[1, 0] = blk[:, 32:]

    out = pl.pallas_call(
        body,
        grid=(R, N // TBLK),
        in_specs=[
            pl.BlockSpec(memory_space=pltpu.SMEM),
            pl.BlockSpec((B, TBLK, H), lambda r, i: (0, i, 0)),
        ],
        out_specs=pl.BlockSpec((2, 1, TBLK, 32), lambda r, i: (0, r, i, 0)),
        out_shape=jax.ShapeDtypeStruct((2, R, N, 32), jnp.float32),
    )(w_rel1, w_bases1)
    return out.reshape(2, R * N, 32)


def _build_t2(xh, w_rel2, w_bases2):
    def body(wr_ref, wb_ref, x_ref, out_ref):
        r = pl.program_id(1)
        xb = jnp.concatenate([x_ref[0], x_ref[1]], axis=1)
        xb = jnp.maximum(xb, 0.0)
        w2r = wr_ref[r, 0] * wb_ref[0] + wr_ref[r, 1] * wb_ref[1]
        y = jnp.dot(xb, w2r, preferred_element_type=jnp.float32,
                    precision=lax.Precision.HIGHEST)
        out_ref[0, 0] = y[:, :32]
        out_ref[1, 0] = y[:, 32:]

    out = pl.pallas_call(
        body,
        grid=(N // TBLK, R),
        in_specs=[
            pl.BlockSpec(memory_space=pltpu.SMEM),
            pl.BlockSpec((B, H, O), lambda i, r: (0, 0, 0)),
            pl.BlockSpec((2, TBLK, 32), lambda i, r: (0, i, 0)),
        ],
        out_specs=pl.BlockSpec((2, 1, TBLK, 32), lambda i, r: (0, r, i, 0)),
        out_shape=jax.ShapeDtypeStruct((2, R, N, 32), jnp.float32),
    )(w_rel2, w_bases2, xh)
    return out.reshape(2, R * N, 32)


@jax.jit
def kernel(edge_rel, edge_src, edge_dst, edge_val,
           w_bases1, w_rel1, w_bases2, w_rel2):
    edge_rel = edge_rel.astype(jnp.int32)
    edge_src = edge_src.astype(jnp.int32)
    edge_dst = edge_dst.astype(jnp.int32)

    g = (edge_rel * N + edge_src).reshape(NBLK, IBN, IB)
    dst = edge_dst.reshape(NBLK, IBN, IB)

    t1 = _build_t1(w_rel1, w_bases1)
    xh = _sc_pass(t1, g, dst, edge_val)
    t2 = _build_t2(xh, w_rel2, w_bases2)
    oh = _sc_pass(t2, g, dst, edge_val)
    return jnp.concatenate([oh[0], oh[1]], axis=1)


# speedup vs baseline: 3.5692x; 1.7222x over previous
import dataclasses
import functools

import jax
import jax.numpy as jnp
from jax import lax
from jax.experimental import pallas as pl
from jax.experimental.pallas import tpu as pltpu
from jax.experimental.pallas import tpu_sc as plsc

N = 50000
R = 8
B = 2
H = 64
O = 64
E = 800000

NS = 16
IB = 100
IBN = 4
BLK = IB * IBN
NBLK = E // BLK
KMAX = NBLK // NS
MU = 8
WBLK = 400
NWB = N // WBLK
KWB = (NWB + NS - 1) // NS


def _sc_pass(table, g, dst, val):
    mesh = plsc.VectorSubcoreMesh(core_axis_name="c", subcore_axis_name="s")

    cp = pltpu.CompilerParams()
    for f, v in (("needs_layout_passes", False), ("use_tc_tiling_on_sc", False)):
        if f in pltpu.CompilerParams.__dataclass_fields__:
            cp = dataclasses.replace(cp, **{f: v})

    @functools.partial(
        pl.kernel,
        compiler_params=cp,
        out_type=jax.ShapeDtypeStruct((2, N, 32), jnp.float32),
        mesh=mesh,
        scratch_types=[
            pltpu.VMEM((3, IBN, IB), jnp.int32),
            pltpu.VMEM((3, IBN, IB), jnp.int32),
            pltpu.VMEM((3, BLK), jnp.float32),
            pltpu.VMEM((2, BLK, 32), jnp.float32),
            pltpu.VMEM_SHARED((N, 32), jnp.float32),
            pltpu.SemaphoreType.DMA((3,)),
            pltpu.SemaphoreType.DMA((2,)),
            pltpu.SemaphoreType.DMA((2,)),
        ],
    )
    def kern(table_hbm, g_hbm, dst_hbm, val_hbm, out_hbm,
             gv, dv, vv, rows, acc, isem, gsem, ssem):
        c = lax.axis_index("c")
        s = lax.axis_index("s")

        zeros16 = jnp.zeros((16,), jnp.float32)

        @pl.loop(0, WBLK)
        def _(i):
            rows[0, i, pl.ds(0, 16)] = zeros16
            rows[0, i, pl.ds(16, 16)] = zeros16

        @pl.loop(0, KWB)
        def _(k):
            b = k * NS + s

            @pl.when(b < NWB)
            def _():
                pltpu.sync_copy(rows.at[0], acc.at[pl.ds(b * WBLK, WBLK)])

        plsc.subcore_barrier()

        def issue_idx(kk, q):
            b = kk * NS + s
            pltpu.async_copy(g_hbm.at[b], gv.at[q], isem.at[q])
            pltpu.async_copy(dst_hbm.at[b], dv.at[q], isem.at[q])
            pltpu.async_copy(val_hbm.at[pl.ds(b * BLK, BLK)], vv.at[q],
                             isem.at[q])

        def wait_idx(kk, q):
            b = kk * NS + s
            pltpu.make_async_copy(g_hbm.at[b], gv.at[q], isem.at[q]).wait()
            pltpu.make_async_copy(dst_hbm.at[b], dv.at[q], isem.at[q]).wait()
            pltpu.make_async_copy(val_hbm.at[pl.ds(b * BLK, BLK)], vv.at[q],
                                  isem.at[q]).wait()

        def issue_gather(q, p):
            for j in range(IBN):
                pltpu.async_copy(table_hbm.at[c].at[gv.at[q].at[j]],
                                 rows.at[p].at[pl.ds(j * IB, IB)], gsem.at[p])

        def wait_gather(q, p):
            for j in range(IBN):
                pltpu.make_async_copy(table_hbm.at[c].at[gv.at[q].at[j]],
                                      rows.at[p].at[pl.ds(j * IB, IB)],
                                      gsem.at[p]).wait()

        def issue_scatter(q, p):
            for j in range(IBN):
                pltpu.async_copy(rows.at[p].at[pl.ds(j * IB, IB)],
                                 acc.at[dv.at[q].at[j]], ssem.at[p], add=True)

        def wait_scatter(q, p):
            for j in range(IBN):
                pltpu.make_async_copy(rows.at[p].at[pl.ds(j * IB, IB)],
                                      acc.at[dv.at[q].at[j]],
                                      ssem.at[p]).wait()

        issue_idx(0, 0)
        issue_idx(1, 1)
        wait_idx(0, 0)
        issue_gather(0, 0)

        @pl.loop(0, KMAX)
        def _(kk):
            p = lax.rem(kk, 2)
            q = lax.rem(kk, 3)
            q1 = lax.rem(kk + 1, 3)
            q2 = lax.rem(kk + 2, 3)

            @pl.when(kk >= 1)
            def _():
                wait_scatter(lax.rem(kk - 1, 3), 1 - p)

            @pl.when(kk + 1 < KMAX)
            def _():
                wait_idx(kk + 1, q1)
                issue_gather(q1, 1 - p)

            @pl.when(kk + 2 < KMAX)
            def _():
                issue_idx(kk + 2, q2)

            wait_gather(q, p)

            @pl.loop(0, BLK, step=MU)
            def _(e0):
                for t in range(MU):
                    e = e0 + t
                    sv = plsc.load_gather(vv.at[q],
                                          [jnp.full((16,), e, jnp.int32)])
                    rows[p, e, pl.ds(0, 16)] = rows[p, e, pl.ds(0, 16)] * sv
                    rows[p, e, pl.ds(16, 16)] = rows[p, e, pl.ds(16, 16)] * sv

            issue_scatter(q, p)

        wait_scatter((KMAX - 1) % 3, (KMAX - 1) % 2)

        plsc.subcore_barrier()

        @pl.loop(0, KWB)
        def _(k):
            b = k * NS + s

            @pl.when(b < NWB)
            def _():
                pltpu.sync_copy(
                    acc.at[pl.ds(b * WBLK, WBLK)],
                    out_hbm.at[c].at[pl.ds(b * WBLK, WBLK)],
                )

    return kern(table, g, dst, val)


TBLK = 2000


def _build_t1(w_rel1, w_bases1):
    def body(wr_ref, wb_ref, out_ref):
        for r in range(R):
            blk = wr_ref[r, 0] * wb_ref[0] + wr_ref[r, 1] * wb_ref[1]
            out_ref[0, r] = blk[:, :32]
            out_ref[1, r] = blk[:, 32:]

    out = pl.pallas_call(
        body,
        grid=(N // TBLK,),
        in_specs=[
            pl.BlockSpec(memory_space=pltpu.SMEM),
            pl.BlockSpec((B, TBLK, H), lambda i: (0, i, 0)),
        ],
        out_specs=pl.BlockSpec((2, R, TBLK, 32), lambda i: (0, 0, i, 0)),
        out_shape=jax.ShapeDtypeStruct((2, R, N, 32), jnp.float32),
    )(w_rel1, w_bases1)
    return out.reshape(2, R * N, 32)


def _build_t2(xh, w_rel2, w_bases2):
    def body(wr_ref, wb_ref, x_ref, out_ref):
        xb = jnp.concatenate([x_ref[0], x_ref[1]], axis=1)
        xb = jnp.maximum(xb, 0.0)
        for r in range(R):
            w2r = wr_ref[r, 0] * wb_ref[0] + wr_ref[r, 1] * wb_ref[1]
            y = jnp.dot(xb, w2r, preferred_element_type=jnp.float32,
                        precision=lax.Precision.HIGHEST)
            out_ref[0, r] = y[:, :32]
            out_ref[1, r] = y[:, 32:]

    out = pl.pallas_call(
        body,
        grid=(N // TBLK,),
        in_specs=[
            pl.BlockSpec(memory_space=pltpu.SMEM),
            pl.BlockSpec((B, H, O), lambda i: (0, 0, 0)),
            pl.BlockSpec((2, TBLK, 32), lambda i: (0, i, 0)),
        ],
        out_specs=pl.BlockSpec((2, R, TBLK, 32), lambda i: (0, 0, i, 0)),
        out_shape=jax.ShapeDtypeStruct((2, R, N, 32), jnp.float32),
    )(w_rel2, w_bases2, xh)
    return out.reshape(2, R * N, 32)


@jax.jit
def kernel(edge_rel, edge_src, edge_dst, edge_val,
           w_bases1, w_rel1, w_bases2, w_rel2):
    edge_rel = edge_rel.astype(jnp.int32)
    edge_src = edge_src.astype(jnp.int32)
    edge_dst = edge_dst.astype(jnp.int32)

    g = (edge_rel * N + edge_src).reshape(NBLK, IBN, IB)
    dst = edge_dst.reshape(NBLK, IBN, IB)

    t1 = _build_t1(w_rel1, w_bases1)
    xh = _sc_pass(t1, g, dst, edge_val)
    t2 = _build_t2(xh, w_rel2, w_bases2)
    oh = _sc_pass(t2, g, dst, edge_val)
    return jnp.concatenate([oh[0], oh[1]], axis=1)
